# Initial kernel scaffold; baseline (speedup 1.0000x reference)
import jax
import jax.numpy as jnp
from jax.experimental import pallas as pl


def kernel(x, idxs, W, b):
    raise NotImplementedError("write your pallas kernel here")


# speedup vs baseline: 2.2092x; 2.2092x over previous
import functools

import jax
import jax.numpy as jnp
from jax import lax
from jax.experimental import pallas as pl
from jax.experimental.pallas import tpu as pltpu
from jax.experimental.pallas import tpu_sc as plsc

N_EXPERTS = 8
TOKENS = 4096
D_IN = 2048
D_OUT = 2048

TM = 512
N_TILES = TOKENS // TM
N_ENTRIES = N_TILES + N_EXPERTS - 1
TM_SHIFT = 9

TN = 1024
NJ = D_OUT // TN

LANES = 16
N_CHUNKS = TOKENS // LANES


def _route_body(idx_hbm, pos_hbm, perm_hbm, offs_hbm, egid_hbm, etile_hbm,
                idx_v, pos_v, perm_v, run_v, t0_v, eexc_v, meta_v, sem):
    wid = lax.axis_index("s") * 2 + lax.axis_index("c")

    @pl.when(wid == 0)
    def _():
        lane = lax.iota(jnp.int32, LANES)
        pltpu.async_copy(idx_hbm, idx_v, sem).wait()

        def count_chunk(c, counts):
            v = idx_v[pl.ds(c * LANES, LANES)]
            for e in range(N_EXPERTS):
                cnt = jnp.sum(jnp.where(v == e, 1, 0))
                counts = counts + jnp.where(lane == e, cnt, 0)
            return counts

        counts = lax.fori_loop(0, N_CHUNKS, count_chunk,
                               jnp.zeros((LANES,), jnp.int32))
        inc = plsc.cumsum(counts)
        starts = inc - counts
        run_v[...] = starts

        def place_chunk(c, _):
            v = idx_v[pl.ds(c * LANES, LANES)]
            tok = c * LANES + lane
            base = plsc.load_gather(run_v, [v])
            newrun = run_v[...]
            pos = base
            for e in range(N_EXPERTS):
                m = v == e
                mwi = jnp.where(m, 1, 0)
                within = plsc.cumsum(mwi)
                cnt = jnp.sum(mwi)
                pos = pos + jnp.where(m, within - 1, 0)
                newrun = newrun + jnp.where(lane == e, cnt, 0)
            run_v[...] = newrun
            pos_v[pl.ds(c * LANES, LANES)] = pos
            plsc.store_scatter(perm_v, [pos], tok)
            return 0

        lax.fori_loop(0, N_CHUNKS, place_chunk, 0)

        valid = (counts > 0) & (lane < N_EXPERTS)
        t0 = lax.shift_right_logical(starts, TM_SHIFT)
        t1 = lax.shift_right_logical(inc - 1, TM_SHIFT)
        ntiles = jnp.where(valid, t1 - t0 + 1, 0)
        einc = plsc.cumsum(ntiles)
        eexc = einc - ntiles
        total = jnp.max(einc)

        t0_v[...] = t0
        eexc_v[...] = eexc
        meta_v[...] = jnp.zeros((LANES,), jnp.int32)
        plsc.store_scatter(meta_v, [eexc], lane, mask=valid)
        gid = plsc.cummax(meta_v[...])
        icl = jnp.minimum(lane, total - 1)
        t0g = plsc.load_gather(t0_v, [gid])
        eexcg = plsc.load_gather(eexc_v, [gid])
        tile = t0g + (icl - eexcg)

        meta_v[...] = starts
        pltpu.async_copy(meta_v, offs_hbm, sem).wait()
        meta_v[...] = gid
        pltpu.async_copy(meta_v, egid_hbm, sem).wait()
        meta_v[...] = tile
        pltpu.async_copy(meta_v, etile_hbm, sem).wait()
        pltpu.async_copy(pos_v, pos_hbm, sem).wait()
        pltpu.async_copy(perm_v, perm_hbm, sem).wait()


def _route(idxs):
    i32 = jnp.int32
    out_type = (
        jax.ShapeDtypeStruct((TOKENS,), i32),
        jax.ShapeDtypeStruct((TOKENS,), i32),
        jax.ShapeDtypeStruct((LANES,), i32),
        jax.ShapeDtypeStruct((LANES,), i32),
        jax.ShapeDtypeStruct((LANES,), i32),
    )
    return pl.kernel(
        _route_body,
        out_type=out_type,
        mesh=plsc.VectorSubcoreMesh(core_axis_name="c", subcore_axis_name="s"),
        compiler_params=pltpu.CompilerParams(needs_layout_passes=False),
        scratch_types=[
            pltpu.VMEM((TOKENS,), i32),
            pltpu.VMEM((TOKENS,), i32),
            pltpu.VMEM((TOKENS,), i32),
            pltpu.VMEM((LANES,), i32),
            pltpu.VMEM((LANES,), i32),
            pltpu.VMEM((LANES,), i32),
            pltpu.VMEM((LANES,), i32),
            pltpu.SemaphoreType.DMA,
        ],
    )(idxs)


_GROWS = 16


def _gather_body(table_hbm, idx_hbm, out_hbm, idx_v, rows_v, sem):
    wid = lax.axis_index("s") * 2 + lax.axis_index("c")
    rows_per_w = TOKENS // 32

    def chunk(c, _):
        base = wid * rows_per_w + c * _GROWS
        pltpu.async_copy(idx_hbm.at[pl.ds(base, _GROWS)], idx_v, sem).wait()
        pltpu.async_copy(table_hbm.at[idx_v], rows_v, sem).wait()
        pltpu.async_copy(rows_v, out_hbm.at[pl.ds(base, _GROWS)], sem).wait()
        return 0

    lax.fori_loop(0, rows_per_w // _GROWS, chunk, 0)


def _gather_rows(table, idx):
    return pl.kernel(
        _gather_body,
        out_type=jax.ShapeDtypeStruct((TOKENS, table.shape[1]), table.dtype),
        mesh=plsc.VectorSubcoreMesh(core_axis_name="c", subcore_axis_name="s"),
        compiler_params=pltpu.CompilerParams(needs_layout_passes=False),
        scratch_types=[
            pltpu.VMEM((_GROWS,), jnp.int32),
            pltpu.VMEM((_GROWS, table.shape[1]), table.dtype),
            pltpu.SemaphoreType.DMA,
        ],
    )(table, idx)


def _gmm_kernel(egid, etile, offs, x_ref, w_ref, b_ref, o_ref):
    i = pl.program_id(1)
    g = egid[i]
    t = etile[i]
    start = offs[g]
    end = jnp.where(g + 1 < N_EXPERTS, offs[g + 1], TOKENS)
    rows = t * TM + lax.broadcasted_iota(jnp.int32, (TM, 1), 0)
    mask = (rows >= start) & (rows < end)
    acc = jnp.dot(x_ref[...], w_ref[0], preferred_element_type=jnp.float32)
    val = jnp.maximum(acc + b_ref[0], 0.0)
    o_ref[...] = jnp.where(mask, val, o_ref[...])


def _gmm(xs, W, b, egid, etile, offs):
    grid_spec = pltpu.PrefetchScalarGridSpec(
        num_scalar_prefetch=3,
        grid=(NJ, N_ENTRIES),
        in_specs=[
            pl.BlockSpec((TM, D_IN), lambda j, i, egid, etile, offs: (etile[i], 0)),
            pl.BlockSpec((1, D_IN, TN), lambda j, i, egid, etile, offs: (egid[i], 0, j)),
            pl.BlockSpec((1, 1, TN), lambda j, i, egid, etile, offs: (egid[i], 0, j)),
        ],
        out_specs=pl.BlockSpec((TM, TN), lambda j, i, egid, etile, offs: (etile[i], j)),
    )
    return pl.pallas_call(
        _gmm_kernel,
        grid_spec=grid_spec,
        out_shape=jax.ShapeDtypeStruct((TOKENS, D_OUT), jnp.float32),
        compiler_params=pltpu.CompilerParams(
            dimension_semantics=("arbitrary", "arbitrary"),
        ),
    )(egid, etile, offs, xs, W, b.reshape(N_EXPERTS, 1, D_OUT))


def kernel(x, idxs, W, b):
    idxs = idxs.astype(jnp.int32)
    pos, perm, offs, egid, etile = _route(idxs)
    xs = _gather_rows(x, perm)
    ys = _gmm(xs, W, b, egid, etile, offs)
    return _gather_rows(ys, pos)
